# baseline (speedup 1.0000x reference)
import jax
import jax.numpy as jnp
from jax import lax
from jax.experimental import pallas as pl
from jax.experimental.pallas import tpu as pltpu
from jax.experimental.pallas import tpu_sc as plsc

V = 1000
B = 4096
T = 50
N = B * T

_info = plsc.get_sparse_core_info()
NC = _info.num_cores
NS = _info.num_subcores
NW = NC * NS
PER_B = B // NW
PER_W = N // NW
NGRP = PER_W // 16
NBLK = PER_W // 128


def _lse_body(table_ref, lse_ref):
    x = table_ref[...]
    m = jnp.max(x, axis=1, keepdims=True)
    s = jnp.sum(jnp.exp(x - m), axis=1, keepdims=True)
    lse_ref[...] = (m + jnp.log(s))[:, 0]


def _loss_body(part_ref, loss_ref):
    loss_ref[...] = jnp.sum(part_ref[...], keepdims=True).reshape(1, 1) / N


def _sc_body(table_hbm, tflat_hbm, ix2d_hbm, ixf_hbm, tgf_hbm, lse_hbm,
             out_hbm, part_hbm,
             ix2d_v, ixf_v, tgf_v, vals_v, rows0_v, rows1_v, lse_v, acc_v,
             gsem0, gsem1, wsem0, wsem1, lsem):
    wid = lax.axis_index("s") * NC + lax.axis_index("c")
    bbase = wid * PER_B
    base = wid * PER_W
    pltpu.sync_copy(ix2d_hbm.at[pl.ds(bbase, PER_B)], ix2d_v)
    pltpu.sync_copy(ixf_hbm.at[pl.ds(base, PER_W)], ixf_v)
    pltpu.sync_copy(tgf_hbm.at[pl.ds(base, PER_W)], tgf_v)
    pltpu.sync_copy(lse_hbm, lse_v)
    acc_v[...] = jnp.zeros((16,), jnp.float32)

    def l1(g, carry):
        off = g * 16
        ixg = ixf_v[pl.ds(off, 16)]
        tgg = tgf_v[pl.ds(off, 16)]
        acc_v[...] = acc_v[...] + plsc.load_gather(lse_v, [ixg])
        ixf_v[pl.ds(off, 16)] = ixg * V + tgg
        return carry

    lax.fori_loop(0, NGRP, l1, 0)

    def l2(k, carry):
        pltpu.make_async_copy(
            tflat_hbm.at[ixf_v.at[pl.ds(k * 128, 128)]],
            vals_v.at[pl.ds(k * 128, 128)], lsem).start()
        return carry

    lax.fori_loop(0, NBLK, l2, 0)

    rows = (rows0_v, rows1_v)
    gsem = (gsem0, gsem1)
    wsem = (wsem0, wsem1)

    def g_copy(c, b):
        return pltpu.make_async_copy(
            table_hbm.at[ix2d_v.at[c]], rows[b], gsem[b])

    def w_copy(c, b):
        return pltpu.make_async_copy(rows[b], out_hbm.at[bbase + c], wsem[b])

    g_copy(0, 0).start()

    def chunk_step(c, b):
        g_copy(c, b).wait()
        w_copy(c, b).start()
        ob = 1 - b

        @pl.when(c + 1 < PER_B)
        def _():
            @pl.when(c >= 1)
            def _():
                w_copy(c - 1, ob).wait()
            g_copy(c + 1, ob).start()

    def body(g, carry):
        chunk_step(2 * g, 0)
        chunk_step(2 * g + 1, 1)
        return carry

    lax.fori_loop(0, PER_B // 2, body, 0)
    w_copy(PER_B - 2, 0).wait()
    w_copy(PER_B - 1, 1).wait()

    def l3(k, carry):
        pltpu.make_async_copy(
            tflat_hbm.at[ixf_v.at[pl.ds(k * 128, 128)]],
            vals_v.at[pl.ds(k * 128, 128)], lsem).wait()
        return carry

    lax.fori_loop(0, NBLK, l3, 0)

    def l4(g, carry):
        acc_v[...] = acc_v[...] - vals_v[pl.ds(g * 16, 16)]
        return carry

    lax.fori_loop(0, NGRP, l4, 0)
    pltpu.sync_copy(acc_v, part_hbm.at[wid])


def kernel(table, ix, target):
    lse = pl.pallas_call(
        _lse_body,
        out_shape=jax.ShapeDtypeStruct((V,), jnp.float32),
    )(table)

    mesh = plsc.VectorSubcoreMesh(core_axis_name="c", subcore_axis_name="s")
    sc = pl.kernel(
        _sc_body,
        mesh=mesh,
        out_type=[
            jax.ShapeDtypeStruct((B, T, V), jnp.float32),
            jax.ShapeDtypeStruct((NW, 16), jnp.float32),
        ],
        scratch_types=[
            pltpu.VMEM((PER_B, T), jnp.int32),
            pltpu.VMEM((PER_W,), jnp.int32),
            pltpu.VMEM((PER_W,), jnp.int32),
            pltpu.VMEM((PER_W,), jnp.float32),
            pltpu.VMEM((T, V), jnp.float32),
            pltpu.VMEM((T, V), jnp.float32),
            pltpu.VMEM((V,), jnp.float32),
            pltpu.VMEM((16,), jnp.float32),
            pltpu.SemaphoreType.DMA,
            pltpu.SemaphoreType.DMA,
            pltpu.SemaphoreType.DMA,
            pltpu.SemaphoreType.DMA,
            pltpu.SemaphoreType.DMA,
        ],
        compiler_params=pltpu.CompilerParams(
            use_tc_tiling_on_sc=False, needs_layout_passes=False
        ),
    )
    tflat = jnp.pad(table.reshape(V * V), (0, 8))
    logits, partials = sc(
        table, tflat, ix, ix.reshape(N), target.reshape(N), lse
    )

    loss2d = pl.pallas_call(
        _loss_body,
        out_shape=jax.ShapeDtypeStruct((1, 1), jnp.float32),
    )(partials)

    return logits, loss2d[0, 0]


# speedup vs baseline: 1.2002x; 1.2002x over previous
import jax
import jax.numpy as jnp
from jax import lax
from jax.experimental import pallas as pl
from jax.experimental.pallas import tpu as pltpu
from jax.experimental.pallas import tpu_sc as plsc

V = 1000
B = 4096
T = 50
N = B * T

_info = plsc.get_sparse_core_info()
NC = _info.num_cores
NS = _info.num_subcores
NW = NC * NS
PER_B = B // NW
PER_W = N // NW
NGRP = PER_W // 16
NBLK = PER_W // 128


def _lse_body(table_ref, lse_ref):
    x = table_ref[...]
    m = jnp.max(x, axis=1, keepdims=True)
    s = jnp.sum(jnp.exp(x - m), axis=1, keepdims=True)
    lse_ref[...] = (m + jnp.log(s))[:, 0]


def _loss_body(part_ref, loss_ref):
    loss_ref[...] = jnp.sum(part_ref[...], keepdims=True).reshape(1, 1) / N


GB = 8
NK = 8


def _split_body(planes_ref, out_ref):
    x = planes_ref[...]
    y = jnp.concatenate([x[k] for k in range(NK)], axis=-1)
    for b_ in range(GB):
        out_ref[b_] = y[b_ * T:(b_ + 1) * T, :V]


def _sc_body(table_hbm, tflat_hbm, ix2d_hbm, ixf_hbm, tgf_hbm, lse_hbm,
             out_hbm, part_hbm,
             ix2d_v, ixf_v, tgf_v, vals_v, rows0_v, rows1_v, lse_v, acc_v,
             gsem0, gsem1, wsem0, wsem1, lsem):
    wid = lax.axis_index("s") * NC + lax.axis_index("c")
    bbase = wid * PER_B
    base = wid * PER_W

    rows = (rows0_v, rows1_v)
    gsem = (gsem0, gsem1)
    wsem = (wsem0, wsem1)

    def g_copy(c, b):
        return pltpu.make_async_copy(
            table_hbm.at[ix2d_v.at[c]], rows[b], gsem[b])

    def w_copies(c, b):
        off = (bbase + c) * T
        res = []
        for k in range(NK - 1):
            res.append(pltpu.make_async_copy(
                rows[b].at[:, pl.ds(k * 128, 128)],
                out_hbm.at[k, pl.ds(off, T)], wsem[b]))
        res.append(pltpu.make_async_copy(
            rows[b].at[:, pl.ds(896, 104)],
            out_hbm.at[NK - 1, pl.ds(off, T), pl.ds(0, 104)], wsem[b]))
        return res

    def w_start(c, b):
        for cp in w_copies(c, b):
            cp.start()

    def w_wait(c, b):
        for cp in w_copies(c, b):
            cp.wait()

    pltpu.sync_copy(ix2d_hbm.at[pl.ds(bbase, PER_B)], ix2d_v)
    g_copy(0, 0).start()
    pltpu.sync_copy(ixf_hbm.at[pl.ds(base, PER_W)], ixf_v)
    pltpu.sync_copy(tgf_hbm.at[pl.ds(base, PER_W)], tgf_v)
    pltpu.sync_copy(lse_hbm, lse_v)
    acc_v[...] = jnp.zeros((16,), jnp.float32)

    def l1(g, carry):
        off = g * 16
        ixg = ixf_v[pl.ds(off, 16)]
        tgg = tgf_v[pl.ds(off, 16)]
        acc_v[...] = acc_v[...] + plsc.load_gather(lse_v, [ixg])
        ixf_v[pl.ds(off, 16)] = ixg * V + tgg
        return carry

    lax.fori_loop(0, NGRP, l1, 0)

    def l2(k, carry):
        pltpu.make_async_copy(
            tflat_hbm.at[ixf_v.at[pl.ds(k * 128, 128)]],
            vals_v.at[pl.ds(k * 128, 128)], lsem).start()
        return carry

    lax.fori_loop(0, NBLK, l2, 0)

    g_copy(0, 0).wait()
    w_start(0, 0)
    g_copy(1, 1).start()

    def body(g, carry):
        c1 = 2 * g + 1
        g_copy(c1, 1).wait()
        w_start(c1, 1)
        w_wait(c1 - 1, 0)
        g_copy(c1 + 1, 0).start()
        c2 = 2 * g + 2
        g_copy(c2, 0).wait()
        w_start(c2, 0)
        w_wait(c2 - 1, 1)
        g_copy(c2 + 1, 1).start()
        return carry

    lax.fori_loop(0, PER_B // 2 - 1, body, 0)
    g_copy(PER_B - 1, 1).wait()
    w_start(PER_B - 1, 1)
    w_wait(PER_B - 2, 0)
    w_wait(PER_B - 1, 1)

    def l3(k, carry):
        pltpu.make_async_copy(
            tflat_hbm.at[ixf_v.at[pl.ds(k * 128, 128)]],
            vals_v.at[pl.ds(k * 128, 128)], lsem).wait()
        return carry

    lax.fori_loop(0, NBLK, l3, 0)

    def l4(g, carry):
        acc_v[...] = acc_v[...] - vals_v[pl.ds(g * 16, 16)]
        return carry

    lax.fori_loop(0, NGRP, l4, 0)
    pltpu.sync_copy(acc_v, part_hbm.at[wid])


def kernel(table, ix, target):
    lse = pl.pallas_call(
        _lse_body,
        out_shape=jax.ShapeDtypeStruct((V,), jnp.float32),
    )(table)

    mesh = plsc.VectorSubcoreMesh(core_axis_name="c", subcore_axis_name="s")
    sc = pl.kernel(
        _sc_body,
        mesh=mesh,
        out_type=[
            jax.ShapeDtypeStruct((NK, N, 128), jnp.float32),
            jax.ShapeDtypeStruct((NW, 16), jnp.float32),
        ],
        scratch_types=[
            pltpu.VMEM((PER_B, T), jnp.int32),
            pltpu.VMEM((PER_W,), jnp.int32),
            pltpu.VMEM((PER_W,), jnp.int32),
            pltpu.VMEM((PER_W,), jnp.float32),
            pltpu.VMEM((T, V), jnp.float32),
            pltpu.VMEM((T, V), jnp.float32),
            pltpu.VMEM((V,), jnp.float32),
            pltpu.VMEM((16,), jnp.float32),
            pltpu.SemaphoreType.DMA,
            pltpu.SemaphoreType.DMA,
            pltpu.SemaphoreType.DMA,
            pltpu.SemaphoreType.DMA,
            pltpu.SemaphoreType.DMA,
        ],
        compiler_params=pltpu.CompilerParams(
            use_tc_tiling_on_sc=False, needs_layout_passes=False
        ),
    )
    tflat = jnp.pad(table.reshape(V * V), (0, 8))
    planes, partials = sc(
        table, tflat, ix, ix.reshape(N), target.reshape(N), lse
    )

    logits = pl.pallas_call(
        _split_body,
        grid=(B // GB,),
        in_specs=[pl.BlockSpec((NK, GB * T, 128), lambda g: (0, g, 0))],
        out_specs=pl.BlockSpec((GB, T, V), lambda g: (g, 0, 0)),
        out_shape=jax.ShapeDtypeStruct((B, T, V), jnp.float32),
    )(planes)

    loss2d = pl.pallas_call(
        _loss_body,
        out_shape=jax.ShapeDtypeStruct((1, 1), jnp.float32),
    )(partials)

    return logits, loss2d[0, 0]


# speedup vs baseline: 1.9697x; 1.6411x over previous
import jax
import jax.numpy as jnp
from jax import lax
from jax.experimental import pallas as pl
from jax.experimental.pallas import tpu as pltpu
from jax.experimental.pallas import tpu_sc as plsc

V = 1000
B = 4096
T = 50
N = B * T

_info = plsc.get_sparse_core_info()
NC = _info.num_cores
NS = _info.num_subcores
NW = NC * NS
PER_B = B // NW
PER_W = N // NW
NGRP = PER_W // 16
NBLK = PER_W // 128
BSUB = 32
NSUB = PER_B // BSUB
NCH = T * NSUB
NK = 8
BB = 1024


def _lse_body(table_ref, lse_ref):
    x = table_ref[...]
    m = jnp.max(x, axis=1, keepdims=True)
    s = jnp.sum(jnp.exp(x - m), axis=1, keepdims=True)
    lse_ref[...] = (m + jnp.log(s))[:, 0]


def _loss_body(part_ref, loss_ref):
    loss_ref[...] = jnp.sum(part_ref[...], keepdims=True).reshape(1, 1) / N


def _split_body(planes_ref, out_ref):
    x = planes_ref[...]
    for k in range(NK - 1):
        out_ref[0, pl.ds(k * 128, 128), :] = jnp.transpose(x[k])
    out_ref[0, pl.ds(896, 104), :] = jnp.transpose(x[NK - 1])[:104]


def _sc_body(table_hbm, tflat_hbm, ixt_hbm, ixf_hbm, tgf_hbm, lse_hbm,
             out_hbm, part_hbm,
             ixt_v, ixf_v, tgf_v, vals_v, rows0_v, rows1_v, lse_v, acc_v,
             gsem0, gsem1, wsem0, wsem1, lsem):
    wid = lax.axis_index("s") * NC + lax.axis_index("c")
    bbase = wid * PER_B
    base = wid * PER_W

    rows = (rows0_v, rows1_v)
    gsem = (gsem0, gsem1)
    wsem = (wsem0, wsem1)

    def g_copy(c, b):
        t = c // NSUB
        jj = c % NSUB
        return pltpu.make_async_copy(
            table_hbm.at[ixt_v.at[t, pl.ds(jj * BSUB, BSUB)]],
            rows[b], gsem[b])

    def w_copies(c, b):
        t = c // NSUB
        jj = c % NSUB
        off = t * B + bbase + jj * BSUB
        res = []
        for k in range(NK - 1):
            res.append(pltpu.make_async_copy(
                rows[b].at[:, pl.ds(k * 128, 128)],
                out_hbm.at[k, pl.ds(off, BSUB)], wsem[b]))
        res.append(pltpu.make_async_copy(
            rows[b].at[:, pl.ds(896, 104)],
            out_hbm.at[NK - 1, pl.ds(off, BSUB), pl.ds(0, 104)], wsem[b]))
        return res

    def w_start(c, b):
        for cp in w_copies(c, b):
            cp.start()

    def w_wait(c, b):
        for cp in w_copies(c, b):
            cp.wait()

    pltpu.sync_copy(ixt_hbm.at[:, pl.ds(bbase, PER_B)], ixt_v)
    g_copy(0, 0).start()
    pltpu.sync_copy(ixf_hbm.at[pl.ds(base, PER_W)], ixf_v)
    pltpu.sync_copy(tgf_hbm.at[pl.ds(base, PER_W)], tgf_v)
    pltpu.sync_copy(lse_hbm, lse_v)
    acc_v[...] = jnp.zeros((16,), jnp.float32)

    def l1(g, carry):
        off = g * 16
        ixg = ixf_v[pl.ds(off, 16)]
        tgg = tgf_v[pl.ds(off, 16)]
        acc_v[...] = acc_v[...] + plsc.load_gather(lse_v, [ixg])
        ixf_v[pl.ds(off, 16)] = ixg * V + tgg
        return carry

    lax.fori_loop(0, NGRP, l1, 0)

    def l2(k, carry):
        pltpu.make_async_copy(
            tflat_hbm.at[ixf_v.at[pl.ds(k * 128, 128)]],
            vals_v.at[pl.ds(k * 128, 128)], lsem).start()
        return carry

    lax.fori_loop(0, NBLK, l2, 0)

    g_copy(0, 0).wait()
    w_start(0, 0)
    g_copy(1, 1).start()

    def body(g, carry):
        c1 = 2 * g + 1
        g_copy(c1, 1).wait()
        w_start(c1, 1)
        w_wait(c1 - 1, 0)
        g_copy(c1 + 1, 0).start()
        c2 = 2 * g + 2
        g_copy(c2, 0).wait()
        w_start(c2, 0)
        w_wait(c2 - 1, 1)
        g_copy(c2 + 1, 1).start()
        return carry

    lax.fori_loop(0, NCH // 2 - 1, body, 0)
    g_copy(NCH - 1, 1).wait()
    w_start(NCH - 1, 1)
    w_wait(NCH - 2, 0)
    w_wait(NCH - 1, 1)

    def l3(k, carry):
        pltpu.make_async_copy(
            tflat_hbm.at[ixf_v.at[pl.ds(k * 128, 128)]],
            vals_v.at[pl.ds(k * 128, 128)], lsem).wait()
        return carry

    lax.fori_loop(0, NBLK, l3, 0)

    def l4(g, carry):
        acc_v[...] = acc_v[...] - vals_v[pl.ds(g * 16, 16)]
        return carry

    lax.fori_loop(0, NGRP, l4, 0)
    pltpu.sync_copy(acc_v, part_hbm.at[wid])


def kernel(table, ix, target):
    lse = pl.pallas_call(
        _lse_body,
        out_shape=jax.ShapeDtypeStruct((V,), jnp.float32),
    )(table)

    mesh = plsc.VectorSubcoreMesh(core_axis_name="c", subcore_axis_name="s")
    sc = pl.kernel(
        _sc_body,
        mesh=mesh,
        out_type=[
            jax.ShapeDtypeStruct((NK, N, 128), jnp.float32),
            jax.ShapeDtypeStruct((NW, 16), jnp.float32),
        ],
        scratch_types=[
            pltpu.VMEM((T, PER_B), jnp.int32),
            pltpu.VMEM((PER_W,), jnp.int32),
            pltpu.VMEM((PER_W,), jnp.int32),
            pltpu.VMEM((PER_W,), jnp.float32),
            pltpu.VMEM((BSUB, V), jnp.float32),
            pltpu.VMEM((BSUB, V), jnp.float32),
            pltpu.VMEM((V,), jnp.float32),
            pltpu.VMEM((16,), jnp.float32),
            pltpu.SemaphoreType.DMA,
            pltpu.SemaphoreType.DMA,
            pltpu.SemaphoreType.DMA,
            pltpu.SemaphoreType.DMA,
            pltpu.SemaphoreType.DMA,
        ],
        compiler_params=pltpu.CompilerParams(
            use_tc_tiling_on_sc=False, needs_layout_passes=False
        ),
    )
    tflat = jnp.pad(table.reshape(V * V), (0, 8))
    planes, partials = sc(
        table, tflat, ix.T, ix.reshape(N), target.reshape(N), lse
    )

    out_tcb = pl.pallas_call(
        _split_body,
        grid=(T, B // BB),
        in_specs=[pl.BlockSpec(
            (NK, BB, 128), lambda t, j: (0, t * (B // BB) + j, 0))],
        out_specs=pl.BlockSpec((1, V, BB), lambda t, j: (t, 0, j)),
        out_shape=jax.ShapeDtypeStruct((T, V, B), jnp.float32),
    )(planes)
    logits = jnp.transpose(out_tcb, (2, 0, 1))

    loss2d = pl.pallas_call(
        _loss_body,
        out_shape=jax.ShapeDtypeStruct((1, 1), jnp.float32),
    )(partials)

    return logits, loss2d[0, 0]


# speedup vs baseline: 2.0085x; 1.0197x over previous
import jax
import jax.numpy as jnp
from jax import lax
from jax.experimental import pallas as pl
from jax.experimental.pallas import tpu as pltpu
from jax.experimental.pallas import tpu_sc as plsc

V = 1000
B = 4096
T = 50
N = B * T

_info = plsc.get_sparse_core_info()
NC = _info.num_cores
NS = _info.num_subcores
NW = NC * NS
PER_B = B // NW
PER_W = N // NW
NGRP = PER_W // 16
NBLK = PER_W // 128
BSUB = 32
NSUB = PER_B // BSUB
NCH = T * NSUB
NK = 8
BB = 2048


def _lse_body(table_ref, lse_ref):
    x = table_ref[...]
    m = jnp.max(x, axis=1, keepdims=True)
    s = jnp.sum(jnp.exp(x - m), axis=1, keepdims=True)
    lse_ref[...] = (m + jnp.log(s))[:, 0]


def _loss_body(part_ref, loss_ref):
    loss_ref[...] = jnp.sum(part_ref[...], keepdims=True).reshape(1, 1) / N


def _split_body(planes_ref, out_ref):
    x = planes_ref[...]
    for k in range(NK - 1):
        out_ref[0, pl.ds(k * 128, 128), :] = jnp.transpose(x[k])
    out_ref[0, pl.ds(896, 104), :] = jnp.transpose(x[NK - 1])[:104]


def _sc_body(table_hbm, tflat_hbm, ixt_hbm, ixf_hbm, tgf_hbm, lse_hbm,
             out_hbm, part_hbm,
             ixt_v, ixf_v, tgf_v, vals_v, rows0_v, rows1_v, lse_v, acc_v,
             gsem0, gsem1, wsem0, wsem1, lsem):
    wid = lax.axis_index("s") * NC + lax.axis_index("c")
    bbase = wid * PER_B
    base = wid * PER_W

    rows = (rows0_v, rows1_v)
    gsem = (gsem0, gsem1)
    wsem = (wsem0, wsem1)

    def g_copy(c, b):
        t = c // NSUB
        jj = c % NSUB
        return pltpu.make_async_copy(
            table_hbm.at[ixt_v.at[t, pl.ds(jj * BSUB, BSUB)]],
            rows[b], gsem[b])

    def w_copies(c, b):
        t = c // NSUB
        jj = c % NSUB
        off = t * B + bbase + jj * BSUB
        res = []
        for k in range(NK - 1):
            res.append(pltpu.make_async_copy(
                rows[b].at[:, pl.ds(k * 128, 128)],
                out_hbm.at[k, pl.ds(off, BSUB)], wsem[b]))
        res.append(pltpu.make_async_copy(
            rows[b].at[:, pl.ds(896, 104)],
            out_hbm.at[NK - 1, pl.ds(off, BSUB), pl.ds(0, 104)], wsem[b]))
        return res

    def w_start(c, b):
        for cp in w_copies(c, b):
            cp.start()

    def w_wait(c, b):
        for cp in w_copies(c, b):
            cp.wait()

    pltpu.sync_copy(ixt_hbm.at[:, pl.ds(bbase, PER_B)], ixt_v)
    g_copy(0, 0).start()
    pltpu.sync_copy(ixf_hbm.at[pl.ds(base, PER_W)], ixf_v)
    pltpu.sync_copy(tgf_hbm.at[pl.ds(base, PER_W)], tgf_v)
    pltpu.sync_copy(lse_hbm, lse_v)
    acc_v[...] = jnp.zeros((16,), jnp.float32)

    def l1(g, carry):
        off = g * 16
        ixg = ixf_v[pl.ds(off, 16)]
        tgg = tgf_v[pl.ds(off, 16)]
        acc_v[...] = acc_v[...] + plsc.load_gather(lse_v, [ixg])
        ixf_v[pl.ds(off, 16)] = ixg * V + tgg
        return carry

    lax.fori_loop(0, NGRP, l1, 0)

    def l2(k, carry):
        pltpu.make_async_copy(
            tflat_hbm.at[ixf_v.at[pl.ds(k * 128, 128)]],
            vals_v.at[pl.ds(k * 128, 128)], lsem).start()
        return carry

    lax.fori_loop(0, NBLK, l2, 0)

    g_copy(0, 0).wait()
    w_start(0, 0)
    g_copy(1, 1).start()

    def body(g, carry):
        c1 = 2 * g + 1
        g_copy(c1, 1).wait()
        w_start(c1, 1)
        w_wait(c1 - 1, 0)
        g_copy(c1 + 1, 0).start()
        c2 = 2 * g + 2
        g_copy(c2, 0).wait()
        w_start(c2, 0)
        w_wait(c2 - 1, 1)
        g_copy(c2 + 1, 1).start()
        return carry

    lax.fori_loop(0, NCH // 2 - 1, body, 0)
    g_copy(NCH - 1, 1).wait()
    w_start(NCH - 1, 1)
    w_wait(NCH - 2, 0)
    w_wait(NCH - 1, 1)

    def l3(k, carry):
        pltpu.make_async_copy(
            tflat_hbm.at[ixf_v.at[pl.ds(k * 128, 128)]],
            vals_v.at[pl.ds(k * 128, 128)], lsem).wait()
        return carry

    lax.fori_loop(0, NBLK, l3, 0)

    def l4(g, carry):
        acc_v[...] = acc_v[...] - vals_v[pl.ds(g * 16, 16)]
        return carry

    lax.fori_loop(0, NGRP, l4, 0)
    pltpu.sync_copy(acc_v, part_hbm.at[wid])


def kernel(table, ix, target):
    lse = pl.pallas_call(
        _lse_body,
        out_shape=jax.ShapeDtypeStruct((V,), jnp.float32),
    )(table)

    mesh = plsc.VectorSubcoreMesh(core_axis_name="c", subcore_axis_name="s")
    sc = pl.kernel(
        _sc_body,
        mesh=mesh,
        out_type=[
            jax.ShapeDtypeStruct((NK, N, 128), jnp.float32),
            jax.ShapeDtypeStruct((NW, 16), jnp.float32),
        ],
        scratch_types=[
            pltpu.VMEM((T, PER_B), jnp.int32),
            pltpu.VMEM((PER_W,), jnp.int32),
            pltpu.VMEM((PER_W,), jnp.int32),
            pltpu.VMEM((PER_W,), jnp.float32),
            pltpu.VMEM((BSUB, V), jnp.float32),
            pltpu.VMEM((BSUB, V), jnp.float32),
            pltpu.VMEM((V,), jnp.float32),
            pltpu.VMEM((16,), jnp.float32),
            pltpu.SemaphoreType.DMA,
            pltpu.SemaphoreType.DMA,
            pltpu.SemaphoreType.DMA,
            pltpu.SemaphoreType.DMA,
            pltpu.SemaphoreType.DMA,
        ],
        compiler_params=pltpu.CompilerParams(
            use_tc_tiling_on_sc=False, needs_layout_passes=False
        ),
    )
    tflat = jnp.pad(table.reshape(V * V), (0, 8))
    planes, partials = sc(
        table, tflat, ix.T, ix.reshape(N), target.reshape(N), lse
    )

    out_tcb = pl.pallas_call(
        _split_body,
        grid=(T, B // BB),
        in_specs=[pl.BlockSpec(
            (NK, BB, 128), lambda t, j: (0, t * (B // BB) + j, 0))],
        out_specs=pl.BlockSpec((1, V, BB), lambda t, j: (t, 0, j)),
        out_shape=jax.ShapeDtypeStruct((T, V, B), jnp.float32),
    )(planes)
    logits = jnp.transpose(out_tcb, (2, 0, 1))

    loss2d = pl.pallas_call(
        _loss_body,
        out_shape=jax.ShapeDtypeStruct((1, 1), jnp.float32),
    )(partials)

    return logits, loss2d[0, 0]
